# Initial kernel scaffold; baseline (speedup 1.0000x reference)
import jax
import jax.numpy as jnp
from jax.experimental import pallas as pl


def kernel(x, csr_idx, W_e1_0, W_e1_1, W_s_0, W_s_1, W_e2_0, W_e2_1):
    raise NotImplementedError("write your pallas kernel here")


# speedup vs baseline: 1.8217x; 1.8217x over previous
import functools

import jax
import jax.numpy as jnp
from jax import lax
from jax.experimental import pallas as pl
from jax.experimental.pallas import tpu as pltpu
from jax.experimental.pallas import tpu_sc as plsc

NW = 32
LANES = 16


def _mlp2_body(x_ref, w0_ref, w1_ref, o_ref):
    h = jnp.maximum(jnp.dot(x_ref[...], w0_ref[...],
                            preferred_element_type=jnp.float32), 0.0)
    o_ref[...] = jnp.maximum(jnp.dot(h, w1_ref[...],
                                     preferred_element_type=jnp.float32), 0.0)


def _set_mlp_body(x_ref, w0_ref, w1_ref, w2_ref, o_ref):
    t = jnp.maximum(jnp.dot(x_ref[...], w0_ref[...],
                            preferred_element_type=jnp.float32), 0.0)
    t = jnp.maximum(jnp.dot(t, w1_ref[...],
                            preferred_element_type=jnp.float32), 0.0)
    o_ref[...] = jnp.dot(t, w2_ref[...], preferred_element_type=jnp.float32)


def _final_body(h_ref, yg_ref, wa_ref, w1_ref, o_ref):
    t = jnp.dot(h_ref[...], wa_ref[...], preferred_element_type=jnp.float32)
    t = jnp.maximum(t + yg_ref[...], 0.0)
    o_ref[...] = jnp.maximum(jnp.dot(t, w1_ref[...],
                                     preferred_element_type=jnp.float32), 0.0)


def _full_spec():
    return pl.BlockSpec((128, 128), lambda i: (0, 0))


def _sread(ref, i):
    return ref[pl.ds(i, LANES)][0]


def _binsearch_seg(csr_v, rows, csr_len):
    pos = jnp.zeros((LANES,), jnp.int32)
    step = csr_len // 2
    while step >= 1:
        cand = pos + step
        vals = plsc.load_gather(csr_v, [cand])
        pos = jnp.where(vals <= rows, cand, pos)
        step //= 2
    return pos


def _make_segmax(n_rows_pad, d, segs_w, csr_len, chunk):
    n_segs_pad = NW * segs_w
    mesh = plsc.VectorSubcoreMesh(core_axis_name="c", subcore_axis_name="s")
    nj = d // LANES

    @functools.partial(
        pl.kernel,
        mesh=mesh,
        out_type=jax.ShapeDtypeStruct((n_segs_pad, d), jnp.float32),
        compiler_params=pltpu.CompilerParams(needs_layout_passes=False),
        scratch_types=[
            pltpu.VMEM((csr_len,), jnp.int32),
            pltpu.VMEM((chunk + LANES, d), jnp.float32),
            pltpu.VMEM((segs_w + 1, d), jnp.float32),
        ],
    )
    def segmax(h_hbm, csr_hbm, xset_hbm, csr_v, buf, xset_v):
        wid = lax.axis_index("s") * 2 + lax.axis_index("c")
        s0 = wid * segs_w
        pltpu.sync_copy(csr_hbm, csr_v)

        zero = jnp.zeros((LANES,), jnp.float32)

        def zbody(i, carry):
            for j in range(nj):
                xset_v[i, pl.ds(j * LANES, LANES)] = zero
            return carry

        lax.fori_loop(0, segs_w + 1, zbody, 0)

        r0 = _sread(csr_v, s0)
        r1 = _sread(csr_v, s0 + segs_w)
        base0 = (r0 // 8) * 8
        nch = (r1 - base0 + chunk - 1) // chunk

        def chunk_body(c, carry):
            base = base0 + c * chunk
            start = jnp.maximum(r0, base)
            end = jnp.minimum(r1, base + chunk)
            pltpu.sync_copy(h_hbm.at[pl.ds(base, chunk)],
                            buf.at[pl.ds(0, chunk)])
            ng = (end - start + LANES - 1) // LANES

            def group_body(g, carry2):
                rows = lax.iota(jnp.int32, LANES) + (start + g * LANES)
                valid = rows < end
                segs = _binsearch_seg(csr_v, rows, csr_len) - s0
                segs = jnp.where(valid, segs, segs_w)
                locs = jnp.where(valid, rows - base, chunk)
                for i in range(LANES):
                    sg = segs[i]
                    lc = locs[i]
                    for j in range(nj):
                        sl = pl.ds(j * LANES, LANES)
                        xset_v[sg, sl] = jnp.maximum(xset_v[sg, sl],
                                                     buf[lc, sl])
                return carry2

            lax.fori_loop(0, ng, group_body, 0)
            return carry

        lax.fori_loop(0, nch, chunk_body, 0)
        pltpu.sync_copy(xset_v.at[pl.ds(0, segs_w)],
                        xset_hbm.at[pl.ds(s0, segs_w)])

    return segmax


def _make_gather(n_rows, d, csr_len, chunk):
    n_chunks = n_rows // chunk
    mesh = plsc.VectorSubcoreMesh(core_axis_name="c", subcore_axis_name="s")
    n_batches = chunk // LANES
    n_steps = csr_len.bit_length() - 1

    @functools.partial(
        pl.kernel,
        mesh=mesh,
        out_type=jax.ShapeDtypeStruct((n_rows, d), jnp.float32),
        compiler_params=pltpu.CompilerParams(needs_layout_passes=False),
        scratch_types=[
            pltpu.VMEM((csr_len,), jnp.int32),
            pltpu.VMEM((chunk,), jnp.int32),
            pltpu.VMEM((chunk, d), jnp.float32),
            pltpu.SemaphoreType.DMA,
        ],
    )
    def gather(y_hbm, csr_hbm, out_hbm, csr_v, idx_v, buf, sem):
        wid = lax.axis_index("s") * 2 + lax.axis_index("c")
        pltpu.sync_copy(csr_hbm, csr_v)
        my_n = (n_chunks - wid + NW - 1) // NW

        def chunk_body(i, carry):
            c = wid + i * NW
            base = c * chunk
            for b in range(n_batches):
                rows = lax.iota(jnp.int32, LANES) + (base + b * LANES)
                pos = jnp.zeros((LANES,), jnp.int32)
                step = csr_len // 2
                while step >= 1:
                    cand = pos + step
                    vals = plsc.load_gather(csr_v, [cand])
                    pos = jnp.where(vals <= rows, cand, pos)
                    step //= 2
                idx_v[pl.ds(b * LANES, LANES)] = pos
            pltpu.async_copy(y_hbm.at[idx_v], buf, sem).wait()
            pltpu.sync_copy(buf, out_hbm.at[pl.ds(base, chunk)])
            return carry

        lax.fori_loop(0, my_n, chunk_body, 0)

    return gather


def kernel(x, csr_idx, W_e1_0, W_e1_1, W_s_0, W_s_1, W_e2_0, W_e2_1):
    n, d = x.shape
    b = csr_idx.shape[0] - 1

    RB = 512
    n_pad = n + RB

    SEGS_W = (b + NW - 1) // NW
    SEGS_W = ((SEGS_W + 7) // 8) * 8
    b_pad = NW * SEGS_W

    CSR_LEN = 1
    while CSR_LEN < b_pad + SEGS_W + 16:
        CSR_LEN *= 2

    csr = csr_idx.astype(jnp.int32)
    csr_pad = jnp.full((CSR_LEN,), jnp.int32(n), dtype=jnp.int32)
    csr_pad = lax.dynamic_update_slice(csr_pad, csr, (0,))

    n_blocks = n // RB
    h = pl.pallas_call(
        _mlp2_body,
        grid=(n_pad // RB,),
        in_specs=[
            pl.BlockSpec((RB, d), lambda i: (jnp.minimum(i, n_blocks - 1), 0)),
            _full_spec(), _full_spec(),
        ],
        out_specs=pl.BlockSpec((RB, d), lambda i: (i, 0)),
        out_shape=jax.ShapeDtypeStruct((n_pad, d), jnp.float32),
    )(x, W_e1_0, W_e1_1)

    segmax = _make_segmax(n_pad, d, SEGS_W, CSR_LEN, 256)
    x_set = segmax(h, csr_pad)

    SB = 1024
    y_set = pl.pallas_call(
        _set_mlp_body,
        grid=(b_pad // SB,),
        in_specs=[pl.BlockSpec((SB, d), lambda i: (i, 0)),
                  _full_spec(), _full_spec(), _full_spec()],
        out_specs=pl.BlockSpec((SB, d), lambda i: (i, 0)),
        out_shape=jax.ShapeDtypeStruct((b_pad, d), jnp.float32),
    )(x_set, W_s_0, W_s_1, W_e2_0[d:])

    gather = _make_gather(n, d, CSR_LEN, 128)
    y_g = gather(y_set, csr_pad)

    out = pl.pallas_call(
        _final_body,
        grid=(n // RB,),
        in_specs=[pl.BlockSpec((RB, d), lambda i: (i, 0)),
                  pl.BlockSpec((RB, d), lambda i: (i, 0)),
                  _full_spec(), _full_spec()],
        out_specs=pl.BlockSpec((RB, d), lambda i: (i, 0)),
        out_shape=jax.ShapeDtypeStruct((n, d), jnp.float32),
    )(h, y_g, W_e2_0[:d], W_e2_1)

    return out


# speedup vs baseline: 2.5146x; 1.3804x over previous
import functools

import jax
import jax.numpy as jnp
from jax import lax
from jax.experimental import pallas as pl
from jax.experimental.pallas import tpu as pltpu
from jax.experimental.pallas import tpu_sc as plsc

NW = 32
LANES = 16


def _mlp2_body(x_ref, w0_ref, w1_ref, o_ref):
    h = jnp.maximum(jnp.dot(x_ref[...], w0_ref[...],
                            preferred_element_type=jnp.float32), 0.0)
    o_ref[...] = jnp.maximum(jnp.dot(h, w1_ref[...],
                                     preferred_element_type=jnp.float32), 0.0)


def _set_mlp_body(x_ref, w0_ref, w1_ref, w2_ref, o_ref):
    t = jnp.maximum(jnp.dot(x_ref[...], w0_ref[...],
                            preferred_element_type=jnp.float32), 0.0)
    t = jnp.maximum(jnp.dot(t, w1_ref[...],
                            preferred_element_type=jnp.float32), 0.0)
    o_ref[...] = jnp.dot(t, w2_ref[...], preferred_element_type=jnp.float32)


def _final_body(h_ref, yg_ref, wa_ref, w1_ref, o_ref):
    t = jnp.dot(h_ref[...], wa_ref[...], preferred_element_type=jnp.float32)
    t = jnp.maximum(t + yg_ref[...], 0.0)
    o_ref[...] = jnp.maximum(jnp.dot(t, w1_ref[...],
                                     preferred_element_type=jnp.float32), 0.0)


def _full_spec():
    return pl.BlockSpec((128, 128), lambda i: (0, 0))


def _sread(ref, i):
    return ref[pl.ds(i, LANES)][0]


def _binsearch_seg(csr_v, rows, span):
    pos = jnp.zeros((LANES,), jnp.int32)
    step = span // 2
    while step >= 1:
        cand = pos + step
        vals = plsc.load_gather(csr_v, [cand])
        pos = jnp.where(vals <= rows, cand, pos)
        step //= 2
    return pos


def _make_segmax(n_rows_pad, d, segs_w, csr_len, chunk):
    n_segs_pad = NW * segs_w
    win = 1
    while win < segs_w + 2:
        win *= 2
    mesh = plsc.VectorSubcoreMesh(core_axis_name="c", subcore_axis_name="s")
    nj = d // LANES

    @functools.partial(
        pl.kernel,
        mesh=mesh,
        out_type=jax.ShapeDtypeStruct((n_segs_pad, d), jnp.float32),
        compiler_params=pltpu.CompilerParams(needs_layout_passes=False),
        scratch_types=[
            pltpu.VMEM((win,), jnp.int32),
            pltpu.VMEM((chunk, d), jnp.float32),
            pltpu.VMEM((chunk, d), jnp.float32),
            pltpu.VMEM((segs_w, d), jnp.float32),
            pltpu.SemaphoreType.DMA,
            pltpu.SemaphoreType.DMA,
        ],
    )
    def segmax(h_hbm, csr_hbm, xset_hbm, csr_v, buf0, buf1, xset_v,
               sem0, sem1):
        wid = lax.axis_index("s") * 2 + lax.axis_index("c")
        s0 = wid * segs_w
        pltpu.sync_copy(csr_hbm.at[pl.ds(s0, win)], csr_v)

        zero = jnp.zeros((LANES,), jnp.float32)

        def zbody(i, carry):
            for j in range(nj):
                xset_v[i, pl.ds(j * LANES, LANES)] = zero
            return carry

        lax.fori_loop(0, segs_w, zbody, 0)

        r0 = _sread(csr_v, 0)
        r1 = _sread(csr_v, segs_w)
        base0 = (r0 // 8) * 8
        nch = jnp.where(r1 > r0, (r1 - base0 + chunk - 1) // chunk, 0)
        last = jnp.maximum(nch - 1, 0)

        def dma_start(c, buf, sem):
            base = base0 + jnp.minimum(c, last) * chunk
            pltpu.async_copy(h_hbm.at[pl.ds(base, chunk)], buf, sem)

        def dma_wait(buf, sem):
            pltpu.make_async_copy(h_hbm.at[pl.ds(0, chunk)], buf, sem).wait()

        def process(c, buf):
            c = jnp.minimum(c, last)
            base = base0 + c * chunk
            start = jnp.maximum(r0, base)
            end = jnp.minimum(r1, base + chunk)
            lanes = lax.iota(jnp.int32, LANES)
            pos = _binsearch_seg(csr_v,
                                 jnp.where(lanes == 1, end - 1, start), win)
            sa = pos[0]
            sb = pos[1]

            def seg_body(s, carry):
                rs = jnp.maximum(_sread(csr_v, s), start)
                re = jnp.minimum(_sread(csr_v, s + 1), end)
                acc = [xset_v[s, pl.ds(j * LANES, LANES)] for j in range(nj)]

                def rbody(r, a):
                    loc = r - base
                    return [jnp.maximum(a[j], buf[loc, pl.ds(j * LANES,
                                                             LANES)])
                            for j in range(nj)]

                acc = lax.fori_loop(rs, re, rbody, acc)
                for j in range(nj):
                    xset_v[s, pl.ds(j * LANES, LANES)] = acc[j]
                return carry

            lax.fori_loop(sa, sb + 1, seg_body, 0)

        dma_start(0, buf0, sem0)
        dma_start(1, buf1, sem1)
        npair = (nch + 1) // 2

        def pair_body(p, carry):
            dma_wait(buf0, sem0)
            process(2 * p, buf0)
            dma_start(2 * p + 2, buf0, sem0)
            dma_wait(buf1, sem1)
            process(2 * p + 1, buf1)
            dma_start(2 * p + 3, buf1, sem1)
            return carry

        lax.fori_loop(0, npair, pair_body, 0)
        dma_wait(buf0, sem0)
        dma_wait(buf1, sem1)
        pltpu.sync_copy(xset_v, xset_hbm.at[pl.ds(s0, segs_w)])

    return segmax


def _make_gather(n_rows, d, csr_len, chunk):
    n_chunks = n_rows // chunk
    mesh = plsc.VectorSubcoreMesh(core_axis_name="c", subcore_axis_name="s")
    n_batches = chunk // LANES
    n_steps = csr_len.bit_length() - 1

    @functools.partial(
        pl.kernel,
        mesh=mesh,
        out_type=jax.ShapeDtypeStruct((n_rows, d), jnp.float32),
        compiler_params=pltpu.CompilerParams(needs_layout_passes=False),
        scratch_types=[
            pltpu.VMEM((csr_len,), jnp.int32),
            pltpu.VMEM((chunk,), jnp.int32),
            pltpu.VMEM((chunk,), jnp.int32),
            pltpu.VMEM((chunk, d), jnp.float32),
            pltpu.VMEM((chunk, d), jnp.float32),
            pltpu.SemaphoreType.DMA,
            pltpu.SemaphoreType.DMA,
            pltpu.SemaphoreType.DMA,
            pltpu.SemaphoreType.DMA,
        ],
    )
    def gather(y_hbm, csr_hbm, out_hbm, csr_v, idx0, idx1, buf0, buf1,
               semg0, semg1, semw0, semw1):
        wid = lax.axis_index("s") * 2 + lax.axis_index("c")
        pltpu.sync_copy(csr_hbm, csr_v)
        my_n = (n_chunks - wid + NW - 1) // NW
        lastk = jnp.maximum(my_n - 1, 0)

        def fill_idx(k, idx_v):
            base = (wid + jnp.minimum(k, lastk) * NW) * chunk
            for b in range(n_batches):
                rows = lax.iota(jnp.int32, LANES) + (base + b * LANES)
                idx_v[pl.ds(b * LANES, LANES)] = _binsearch_seg(
                    csr_v, rows, csr_len)
            return base

        def pair_body(p, carry):
            b0 = fill_idx(2 * p, idx0)
            pltpu.async_copy(y_hbm.at[idx0], buf0, semg0)
            b1 = fill_idx(2 * p + 1, idx1)
            pltpu.async_copy(y_hbm.at[idx1], buf1, semg1)
            pltpu.make_async_copy(y_hbm.at[idx0], buf0, semg0).wait()
            pltpu.async_copy(buf0, out_hbm.at[pl.ds(b0, chunk)], semw0)
            pltpu.make_async_copy(y_hbm.at[idx1], buf1, semg1).wait()
            pltpu.async_copy(buf1, out_hbm.at[pl.ds(b1, chunk)], semw1)
            pltpu.make_async_copy(buf0, out_hbm.at[pl.ds(b0, chunk)],
                                  semw0).wait()
            pltpu.make_async_copy(buf1, out_hbm.at[pl.ds(b1, chunk)],
                                  semw1).wait()
            return carry

        lax.fori_loop(0, (my_n + 1) // 2, pair_body, 0)

    return gather


def kernel(x, csr_idx, W_e1_0, W_e1_1, W_s_0, W_s_1, W_e2_0, W_e2_1):
    n, d = x.shape
    b = csr_idx.shape[0] - 1

    RB = 512
    n_pad = n + RB

    SEGS_W = (b + NW - 1) // NW
    SEGS_W = ((SEGS_W + 7) // 8) * 8
    b_pad = NW * SEGS_W

    CSR_LEN = 1
    while CSR_LEN < b_pad + SEGS_W + 16:
        CSR_LEN *= 2

    csr = csr_idx.astype(jnp.int32)
    csr_pad = jnp.full((CSR_LEN,), jnp.int32(n), dtype=jnp.int32)
    csr_pad = lax.dynamic_update_slice(csr_pad, csr, (0,))

    n_blocks = n // RB
    h = pl.pallas_call(
        _mlp2_body,
        grid=(n_pad // RB,),
        in_specs=[
            pl.BlockSpec((RB, d), lambda i: (jnp.minimum(i, n_blocks - 1), 0)),
            _full_spec(), _full_spec(),
        ],
        out_specs=pl.BlockSpec((RB, d), lambda i: (i, 0)),
        out_shape=jax.ShapeDtypeStruct((n_pad, d), jnp.float32),
    )(x, W_e1_0, W_e1_1)

    segmax = _make_segmax(n_pad, d, SEGS_W, CSR_LEN, 256)
    x_set = segmax(h, csr_pad)

    SB = 1024
    y_set = pl.pallas_call(
        _set_mlp_body,
        grid=(b_pad // SB,),
        in_specs=[pl.BlockSpec((SB, d), lambda i: (i, 0)),
                  _full_spec(), _full_spec(), _full_spec()],
        out_specs=pl.BlockSpec((SB, d), lambda i: (i, 0)),
        out_shape=jax.ShapeDtypeStruct((b_pad, d), jnp.float32),
    )(x_set, W_s_0, W_s_1, W_e2_0[d:])

    gather = _make_gather(n, d, CSR_LEN, 128)
    y_g = gather(y_set, csr_pad)

    out = pl.pallas_call(
        _final_body,
        grid=(n // RB,),
        in_specs=[pl.BlockSpec((RB, d), lambda i: (i, 0)),
                  pl.BlockSpec((RB, d), lambda i: (i, 0)),
                  _full_spec(), _full_spec()],
        out_specs=pl.BlockSpec((RB, d), lambda i: (i, 0)),
        out_shape=jax.ShapeDtypeStruct((n, d), jnp.float32),
    )(h, y_g, W_e2_0[:d], W_e2_1)

    return out
